# Initial kernel scaffold; baseline (speedup 1.0000x reference)
import jax
import jax.numpy as jnp
from jax.experimental import pallas as pl


def kernel(x, emb1, conv1_w, conv1_b, glob_W, glob_b, conv2_w, conv2_b, loc_w, loc_b, conv3_w, conv3_b, conv4_w, conv4_b, emb2, fc1_W, fc1_b, head_W, head_b):
    raise NotImplementedError("write your pallas kernel here")


# speedup vs baseline: 4.8058x; 4.8058x over previous
import functools

import jax
import jax.numpy as jnp
from jax import lax
from jax.experimental import pallas as pl
from jax.experimental.pallas import tpu as pltpu
from jax.experimental.pallas import tpu_sc as plsc

B, L, V, D, F, H, NOUT, K = 1024, 200, 1000000, 32, 100, 100, 10, 20

NC, NS = 2, 16
NW = NC * NS
TOTAL = B * L
PER_W = TOTAL // NW
SLICE = 128
GROUP = 10
NSLICE = PER_W // SLICE
NGROUP = NSLICE // GROUP


def _gather_body(t1_hbm, t2_hbm, idx_hbm, o1_hbm, o2_hbm,
                 idx_v, rows_v, sem):
    wid = lax.axis_index("s") * NC + lax.axis_index("c")
    pltpu.sync_copy(idx_hbm.at[wid], idx_v)

    def one_table(tab, out):
        def group(g, _):
            for j in range(GROUP):
                pltpu.async_copy(
                    tab.at[idx_v.at[g * GROUP + j]],
                    rows_v.at[pl.ds(j * SLICE, SLICE)], sem)
            base = wid * PER_W + g * GROUP * SLICE
            pltpu.make_async_copy(
                out.at[pl.ds(base, GROUP * SLICE)], rows_v, sem).wait()
            pltpu.sync_copy(rows_v,
                            out.at[pl.ds(base, GROUP * SLICE)])
            return 0
        lax.fori_loop(0, NGROUP, group, 0, unroll=False)

    one_table(t1_hbm, o1_hbm)
    one_table(t2_hbm, o2_hbm)


@jax.jit
def _sc_gather(emb1, emb2, idx):
    mesh = plsc.VectorSubcoreMesh(core_axis_name="c", subcore_axis_name="s")
    out_t = (jax.ShapeDtypeStruct((TOTAL, D), jnp.float32),
             jax.ShapeDtypeStruct((TOTAL, D), jnp.float32))
    return pl.kernel(
        _gather_body,
        out_type=out_t,
        mesh=mesh,
        compiler_params=pltpu.CompilerParams(use_tc_tiling_on_sc=False),
        scratch_types=[
            pltpu.VMEM((NSLICE, SLICE), jnp.int32),
            pltpu.VMEM((GROUP * SLICE, D), jnp.float32),
            pltpu.SemaphoreType.DMA,
        ],
    )(emb1, emb2, idx)


BB = 16
NEG = -1e30


def _bdot(a, b):
    return jnp.dot(a.astype(jnp.bfloat16), b.astype(jnp.bfloat16),
                   preferred_element_type=jnp.float32)

def _dense_body(e1_ref, e2_ref, w1t_ref, b1_ref, gwt_ref, gb_ref,
                w2t_ref, b2_ref, wlt_ref, bl_ref, w3t_ref,
                b3_ref, w4_ref, f1t_ref, f1b_ref, hwt_ref,
                hb_ref, o_ref, t_ref, epad, hpad, lpad):
    BL = BB * L
    epad[:, 0:1, :] = jnp.zeros((BB, 1, D), jnp.float32)
    epad[:, L + 1:L + 2, :] = jnp.zeros((BB, 1, D), jnp.float32)
    epad[:, 1:L + 1, :] = e1_ref[...]
    ecat = jnp.concatenate(
        [epad[:, t:t + L, :].reshape(BL, D) for t in range(3)], axis=1)
    h = jnp.maximum(_bdot(ecat, w1t_ref[...]) + b1_ref[...], 0.0)

    g = jnp.sum(h.reshape(BB, L, F), axis=1) * (1.0 / L)
    g = jnp.maximum(_bdot(g, gwt_ref[...])
                    + gb_ref[...], 0.0)

    hpad[:, 0:1, :] = jnp.zeros((BB, 1, F), jnp.float32)
    hpad[:, L + 1:L + 2, :] = jnp.zeros((BB, 1, F), jnp.float32)
    hpad[:, 1:L + 1, :] = h.reshape(BB, L, F)
    hcat = jnp.concatenate(
        [hpad[:, t:t + L, :].reshape(BL, F) for t in range(3)], axis=1)
    loc = jnp.maximum(_bdot(hcat, w2t_ref[...]) + b2_ref[...], 0.0)

    lpad[:, 0:1, :] = jnp.zeros((BB, 1, H), jnp.float32)
    lpad[:, L + 1:L + 2, :] = jnp.zeros((BB, 1, H), jnp.float32)
    lpad[:, 1:L + 1, :] = loc.reshape(BB, L, H)
    lcat = jnp.concatenate(
        [lpad[:, t:t + L, :].reshape(BL, H) for t in range(3)], axis=1)
    loc2 = jnp.maximum(_bdot(lcat, wlt_ref[...]) + bl_ref[...], 0.0)

    gb = jnp.broadcast_to(g[:, None, :], (BB, L, H)).reshape(BL, H)
    zcat = jnp.concatenate([gb, loc2], axis=1)
    z = _bdot(zcat, w3t_ref[...]) + b3_ref[...]
    z = jnp.maximum(z, 0.0).reshape(BB, L, F)
    zb = z.astype(jnp.bfloat16).astype(jnp.float32)
    w4b = w4_ref[...].astype(jnp.bfloat16).astype(jnp.float32)
    logits = jnp.sum(zb * w4b[None, :, :], axis=2)

    cols = lax.broadcasted_iota(jnp.int32, (BB, L), 1)
    work = logits
    for _ in range(K - 1):
        m = jnp.max(work, axis=1, keepdims=True)
        hit = work >= m
        pos = jnp.min(jnp.where(hit, cols, L), axis=1, keepdims=True)
        work = jnp.where(cols == pos, NEG, work)
    thr = jnp.max(work, axis=1, keepdims=True)
    tmask = (logits >= thr).astype(jnp.float32)
    t_ref[...] = tmask

    op = jnp.sum(e2_ref[...] * tmask[:, :, None], axis=1) * (1.0 / L)
    op = jnp.maximum(_bdot(op, f1t_ref[...])
                     + f1b_ref[...], 0.0)
    lg = _bdot(op, hwt_ref[...]) \
        + hb_ref[...]
    lg = lg - jnp.max(lg, axis=1, keepdims=True)
    ex = jnp.exp(lg)
    o_ref[...] = ex / jnp.sum(ex, axis=1, keepdims=True)


def _full(spec_shape):
    return pl.BlockSpec(spec_shape, lambda i: (0,) * len(spec_shape))


@jax.jit
def _tc_dense(e1, e2, w1t, b1, gwt, gb, w2t, b2, wlt, bl,
              w3t, b3, w4, f1t, f1b, hwt, hb):
    grid = (B // BB,)
    in_specs = [
        pl.BlockSpec((BB, L, D), lambda i: (i, 0, 0)),
        pl.BlockSpec((BB, L, D), lambda i: (i, 0, 0)),
        _full((3 * D, F)), _full((1, F)), _full((F, H)), _full((1, H)),
        _full((3 * F, H)), _full((1, H)), _full((3 * H, H)), _full((1, H)),
        _full((2 * H, F)), _full((1, F)), _full((1, F)),
        _full((D, H)), _full((1, H)), _full((H, NOUT)),
        _full((1, NOUT)),
    ]
    out_specs = (pl.BlockSpec((BB, NOUT), lambda i: (i, 0)),
                 pl.BlockSpec((BB, L), lambda i: (i, 0)))
    return pl.pallas_call(
        _dense_body,
        grid=grid,
        in_specs=in_specs,
        out_specs=out_specs,
        out_shape=(jax.ShapeDtypeStruct((B, NOUT), jnp.float32),
                   jax.ShapeDtypeStruct((B, L), jnp.float32)),
        scratch_shapes=[
            pltpu.VMEM((BB, L + 2, D), jnp.float32),
            pltpu.VMEM((BB, L + 2, F), jnp.float32),
            pltpu.VMEM((BB, L + 2, H), jnp.float32),
        ],
    )(e1, e2, w1t, b1, gwt, gb, w2t, b2, wlt, bl,
      w3t, b3, w4, f1t, f1b, hwt, hb)


def kernel(x, emb1, conv1_w, conv1_b, glob_W, glob_b, conv2_w, conv2_b,
           loc_w, loc_b, conv3_w, conv3_b, conv4_w, conv4_b, emb2,
           fc1_W, fc1_b, head_W, head_b):
    idx = x.astype(jnp.int32).reshape(NW, NSLICE, SLICE)
    e1, e2 = _sc_gather(emb1, emb2, idx)
    e1 = e1.reshape(B, L, D)
    e2 = e2.reshape(B, L, D)

    w1t = jnp.transpose(conv1_w, (2, 1, 0)).reshape(3 * D, F)
    w2t = jnp.transpose(conv2_w, (2, 1, 0)).reshape(3 * F, H)
    wlt = jnp.transpose(loc_w, (2, 1, 0)).reshape(3 * H, H)
    w3t = jnp.transpose(conv3_w[:, :, 0])
    w4 = conv4_w[0, :, 0][None, :]
    del conv4_b
    o, t = _tc_dense(e1, e2, w1t, conv1_b[None, :], glob_W.T,
                     glob_b[None, :], w2t, conv2_b[None, :], wlt,
                     loc_b[None, :], w3t, conv3_b[None, :], w4,
                     fc1_W.T, fc1_b[None, :], head_W.T,
                     head_b[None, :])
    return o, t


# speedup vs baseline: 5.5024x; 1.1449x over previous
import functools

import jax
import jax.numpy as jnp
from jax import lax
from jax.experimental import pallas as pl
from jax.experimental.pallas import tpu as pltpu
from jax.experimental.pallas import tpu_sc as plsc

B, L, V, D, F, H, NOUT, K = 1024, 200, 1000000, 32, 100, 100, 10, 20

NC, NS = 2, 16
NW = NC * NS
TOTAL = B * L
PER_W = TOTAL // NW
SLICE = 128
GROUP = 10
NSLICE = PER_W // SLICE
NGROUP = NSLICE // GROUP


def _gather_body(t1_hbm, t2_hbm, idx_hbm, o1_hbm, o2_hbm,
                 idx_v, rows_v, sem):
    wid = lax.axis_index("s") * NC + lax.axis_index("c")
    pltpu.sync_copy(idx_hbm.at[wid], idx_v)

    def one_table(tab, out):
        def group(g, _):
            for j in range(GROUP):
                pltpu.async_copy(
                    tab.at[idx_v.at[g * GROUP + j]],
                    rows_v.at[pl.ds(j * SLICE, SLICE)], sem)
            base = wid * PER_W + g * GROUP * SLICE
            pltpu.make_async_copy(
                out.at[pl.ds(base, GROUP * SLICE)], rows_v, sem).wait()
            pltpu.sync_copy(rows_v,
                            out.at[pl.ds(base, GROUP * SLICE)])
            return 0
        lax.fori_loop(0, NGROUP, group, 0, unroll=False)

    one_table(t1_hbm, o1_hbm)
    one_table(t2_hbm, o2_hbm)


@jax.jit
def _sc_gather(emb1, emb2, idx):
    mesh = plsc.VectorSubcoreMesh(core_axis_name="c", subcore_axis_name="s")
    out_t = (jax.ShapeDtypeStruct((TOTAL, D), jnp.float32),
             jax.ShapeDtypeStruct((TOTAL, D), jnp.float32))
    return pl.kernel(
        _gather_body,
        out_type=out_t,
        mesh=mesh,
        compiler_params=pltpu.CompilerParams(use_tc_tiling_on_sc=False),
        scratch_types=[
            pltpu.VMEM((NSLICE, SLICE), jnp.int32),
            pltpu.VMEM((GROUP * SLICE, D), jnp.float32),
            pltpu.SemaphoreType.DMA,
        ],
    )(emb1, emb2, idx)


BB = 64
NEG = -1e30


def _bdot(a, b):
    return jnp.dot(a.astype(jnp.bfloat16), b.astype(jnp.bfloat16),
                   preferred_element_type=jnp.float32)

def _dense_body(e1_ref, e2_ref, w1t_ref, b1_ref, gwt_ref, gb_ref,
                w2t_ref, b2_ref, wlt_ref, bl_ref, w3t_ref,
                b3_ref, w4_ref, f1t_ref, f1b_ref, hwt_ref,
                hb_ref, o_ref, t_ref, epad, hpad, lpad):
    BL = BB * L
    epad[:, 0:1, :] = jnp.zeros((BB, 1, D), jnp.float32)
    epad[:, L + 1:L + 2, :] = jnp.zeros((BB, 1, D), jnp.float32)
    epad[:, 1:L + 1, :] = e1_ref[...]
    ecat = jnp.concatenate(
        [epad[:, t:t + L, :].reshape(BL, D) for t in range(3)], axis=1)
    h = jnp.maximum(_bdot(ecat, w1t_ref[...]) + b1_ref[...], 0.0)

    g = jnp.sum(h.reshape(BB, L, F), axis=1) * (1.0 / L)
    g = jnp.maximum(_bdot(g, gwt_ref[...])
                    + gb_ref[...], 0.0)

    hpad[:, 0:1, :] = jnp.zeros((BB, 1, F), jnp.float32)
    hpad[:, L + 1:L + 2, :] = jnp.zeros((BB, 1, F), jnp.float32)
    hpad[:, 1:L + 1, :] = h.reshape(BB, L, F)
    hcat = jnp.concatenate(
        [hpad[:, t:t + L, :].reshape(BL, F) for t in range(3)], axis=1)
    loc = jnp.maximum(_bdot(hcat, w2t_ref[...]) + b2_ref[...], 0.0)

    lpad[:, 0:1, :] = jnp.zeros((BB, 1, H), jnp.float32)
    lpad[:, L + 1:L + 2, :] = jnp.zeros((BB, 1, H), jnp.float32)
    lpad[:, 1:L + 1, :] = loc.reshape(BB, L, H)
    lcat = jnp.concatenate(
        [lpad[:, t:t + L, :].reshape(BL, H) for t in range(3)], axis=1)
    loc2 = jnp.maximum(_bdot(lcat, wlt_ref[...]) + bl_ref[...], 0.0)

    gb = jnp.broadcast_to(g[:, None, :], (BB, L, H)).reshape(BL, H)
    zcat = jnp.concatenate([gb, loc2], axis=1)
    z = _bdot(zcat, w3t_ref[...]) + b3_ref[...]
    z = jnp.maximum(z, 0.0).reshape(BB, L, F)
    zb = z.astype(jnp.bfloat16).astype(jnp.float32)
    w4b = w4_ref[...].astype(jnp.bfloat16).astype(jnp.float32)
    logits = jnp.sum(zb * w4b[None, :, :], axis=2)

    cols = lax.broadcasted_iota(jnp.int32, (BB, L), 1)
    work = logits
    for _ in range(K - 1):
        m = jnp.max(work, axis=1, keepdims=True)
        hit = work >= m
        pos = jnp.min(jnp.where(hit, cols, L), axis=1, keepdims=True)
        work = jnp.where(cols == pos, NEG, work)
    thr = jnp.max(work, axis=1, keepdims=True)
    tmask = (logits >= thr).astype(jnp.float32)
    t_ref[...] = tmask

    op = jnp.sum(e2_ref[...] * tmask[:, :, None], axis=1) * (1.0 / L)
    op = jnp.maximum(_bdot(op, f1t_ref[...])
                     + f1b_ref[...], 0.0)
    lg = _bdot(op, hwt_ref[...]) \
        + hb_ref[...]
    lg = lg - jnp.max(lg, axis=1, keepdims=True)
    ex = jnp.exp(lg)
    o_ref[...] = ex / jnp.sum(ex, axis=1, keepdims=True)


def _full(spec_shape):
    return pl.BlockSpec(spec_shape, lambda i: (0,) * len(spec_shape))


@jax.jit
def _tc_dense(e1, e2, w1t, b1, gwt, gb, w2t, b2, wlt, bl,
              w3t, b3, w4, f1t, f1b, hwt, hb):
    grid = (B // BB,)
    in_specs = [
        pl.BlockSpec((BB, L, D), lambda i: (i, 0, 0)),
        pl.BlockSpec((BB, L, D), lambda i: (i, 0, 0)),
        _full((3 * D, F)), _full((1, F)), _full((F, H)), _full((1, H)),
        _full((3 * F, H)), _full((1, H)), _full((3 * H, H)), _full((1, H)),
        _full((2 * H, F)), _full((1, F)), _full((1, F)),
        _full((D, H)), _full((1, H)), _full((H, NOUT)),
        _full((1, NOUT)),
    ]
    out_specs = (pl.BlockSpec((BB, NOUT), lambda i: (i, 0)),
                 pl.BlockSpec((BB, L), lambda i: (i, 0)))
    return pl.pallas_call(
        _dense_body,
        grid=grid,
        in_specs=in_specs,
        out_specs=out_specs,
        out_shape=(jax.ShapeDtypeStruct((B, NOUT), jnp.float32),
                   jax.ShapeDtypeStruct((B, L), jnp.float32)),
        scratch_shapes=[
            pltpu.VMEM((BB, L + 2, D), jnp.float32),
            pltpu.VMEM((BB, L + 2, F), jnp.float32),
            pltpu.VMEM((BB, L + 2, H), jnp.float32),
        ],
    )(e1, e2, w1t, b1, gwt, gb, w2t, b2, wlt, bl,
      w3t, b3, w4, f1t, f1b, hwt, hb)


def kernel(x, emb1, conv1_w, conv1_b, glob_W, glob_b, conv2_w, conv2_b,
           loc_w, loc_b, conv3_w, conv3_b, conv4_w, conv4_b, emb2,
           fc1_W, fc1_b, head_W, head_b):
    idx = x.astype(jnp.int32).reshape(NW, NSLICE, SLICE)
    e1, e2 = _sc_gather(emb1, emb2, idx)
    e1 = e1.reshape(B, L, D)
    e2 = e2.reshape(B, L, D)

    w1t = jnp.transpose(conv1_w, (2, 1, 0)).reshape(3 * D, F)
    w2t = jnp.transpose(conv2_w, (2, 1, 0)).reshape(3 * F, H)
    wlt = jnp.transpose(loc_w, (2, 1, 0)).reshape(3 * H, H)
    w3t = jnp.transpose(conv3_w[:, :, 0])
    w4 = conv4_w[0, :, 0][None, :]
    del conv4_b
    o, t = _tc_dense(e1, e2, w1t, conv1_b[None, :], glob_W.T,
                     glob_b[None, :], w2t, conv2_b[None, :], wlt,
                     loc_b[None, :], w3t, conv3_b[None, :], w4,
                     fc1_W.T, fc1_b[None, :], head_W.T,
                     head_b[None, :])
    return o, t
